# Initial kernel scaffold; baseline (speedup 1.0000x reference)
import jax
import jax.numpy as jnp
from jax.experimental import pallas as pl


def kernel(x, edge_index, join_index, W1, b1, W2, b2, W3, b3):
    raise NotImplementedError("write your pallas kernel here")


# speedup vs baseline: 14.0151x; 14.0151x over previous
import functools

import jax
import jax.numpy as jnp
from jax import lax
from jax.experimental import pallas as pl
from jax.experimental.pallas import tpu as pltpu
from jax.experimental.pallas import tpu_sc as plsc

N = 10000
D = 128
NPAD = 10240
NC = 2
NS = 16
NW = NC * NS
CHUNK = 128
E = 320000
CH = -(-E // (NW * CHUNK))
EPAD = NW * CH * CHUNK
BJ = 1024
BJW = BJ // NW
BM = 512
GRID = NPAD // BM

_mesh = plsc.VectorSubcoreMesh(
    core_axis_name="c", subcore_axis_name="s", num_cores=NC, num_subcores=NS)


def _sc_deg_body(dst_hbm, ones_hbm, zeros_hbm, out_hbm, idx_d, ones_v, accum, sem):
    c = lax.axis_index("c")
    s = lax.axis_index("s")
    wid = c * NS + s
    rows = NPAD // NS
    r0 = s * rows
    pltpu.sync_copy(zeros_hbm.at[pl.ds(r0, rows)], accum.at[pl.ds(r0, rows)])
    pltpu.sync_copy(ones_hbm, ones_v)
    pltpu.sync_copy(dst_hbm.at[wid], idx_d)
    plsc.subcore_barrier()

    def chunk(j, carry):
        pltpu.sync_copy(ones_v, accum.at[idx_d.at[j]], add=True)
        return carry

    lax.fori_loop(0, CH, chunk, 0)
    plsc.subcore_barrier()
    pltpu.sync_copy(accum.at[pl.ds(r0, rows)], out_hbm.at[c, pl.ds(r0, rows)])


def _sc_deg(dst_idx, ones, zeros):
    return pl.kernel(
        _sc_deg_body,
        out_type=jax.ShapeDtypeStruct((NC, NPAD, 16), jnp.float32),
        mesh=_mesh,
        scratch_types=[
            pltpu.VMEM((CH, CHUNK), jnp.int32),
            pltpu.VMEM((CHUNK, 16), jnp.float32),
            pltpu.VMEM_SHARED((NPAD, 16), jnp.float32),
            pltpu.SemaphoreType.DMA,
        ],
    )(dst_idx, ones, zeros)


def _sc_scatter_body(y_hbm, src_hbm, dst_hbm, zeros_hbm, out_hbm,
                          idx_s, idx_d, buf, accum, sem):
    c = lax.axis_index("c")
    s = lax.axis_index("s")
    wid = c * NS + s
    rows = NPAD // NS
    r0 = s * rows
    pltpu.sync_copy(zeros_hbm.at[pl.ds(r0, rows)], accum.at[pl.ds(r0, rows)])
    pltpu.sync_copy(src_hbm.at[wid], idx_s)
    pltpu.sync_copy(dst_hbm.at[wid], idx_d)
    plsc.subcore_barrier()

    def chunk(j, carry):
        pltpu.async_copy(y_hbm.at[idx_s.at[j]], buf, sem).wait()
        pltpu.sync_copy(buf, accum.at[idx_d.at[j]], add=True)
        return carry

    lax.fori_loop(0, CH, chunk, 0)
    plsc.subcore_barrier()
    pltpu.sync_copy(accum.at[pl.ds(r0, rows)], out_hbm.at[c, pl.ds(r0, rows)])


def _sc_scatter(y, src_idx, dst_idx, zeros):
    return pl.kernel(
        _sc_scatter_body,
        out_type=jax.ShapeDtypeStruct((NC, NPAD, D), jnp.float32),
        mesh=_mesh,
        scratch_types=[
            pltpu.VMEM((CH, CHUNK), jnp.int32),
            pltpu.VMEM((CH, CHUNK), jnp.int32),
            pltpu.VMEM((CHUNK, D), jnp.float32),
            pltpu.VMEM_SHARED((NPAD, D), jnp.float32),
            pltpu.SemaphoreType.DMA,
        ],
    )(y, src_idx, dst_idx, zeros)


def _sc_join_body(z_hbm, jidx_hbm, out_hbm, jidx_v, buf, sem):
    c = lax.axis_index("c")
    s = lax.axis_index("s")
    wid = c * NS + s
    pltpu.sync_copy(jidx_hbm.at[wid], jidx_v)
    pltpu.async_copy(z_hbm.at[jidx_v], buf, sem).wait()
    pltpu.sync_copy(buf, out_hbm.at[pl.ds(wid * BJW, BJW)])


def _sc_join(z16, jidx):
    return pl.kernel(
        _sc_join_body,
        out_type=jax.ShapeDtypeStruct((BJ, D), jnp.float32),
        mesh=_mesh,
        scratch_types=[
            pltpu.VMEM((BJW,), jnp.int32),
            pltpu.VMEM((BJW, D), jnp.float32),
            pltpu.SemaphoreType.DMA,
        ],
    )(z16, jidx)


def _tc_k1_body(deg_ref, x_ref, y1_ref, dinv_ref):
    deg = deg_ref[0] + deg_ref[1] + 1.0
    dinv = lax.rsqrt(deg)
    dinv_ref[...] = dinv
    y1_ref[...] = x_ref[...] * dinv[:, 0:1]


def _tc_k1(deg_part, x_pad):
    return pl.pallas_call(
        _tc_k1_body,
        grid=(GRID,),
        in_specs=[
            pl.BlockSpec((NC, BM, 16), lambda m: (0, m, 0)),
            pl.BlockSpec((BM, D), lambda m: (m, 0)),
        ],
        out_specs=[
            pl.BlockSpec((BM, D), lambda m: (m, 0)),
            pl.BlockSpec((BM, 16), lambda m: (m, 0)),
        ],
        out_shape=[
            jax.ShapeDtypeStruct((NPAD, D), jnp.float32),
            jax.ShapeDtypeStruct((NPAD, 16), jnp.float32),
        ],
    )(deg_part, x_pad)


def _tc_k2_body(p_ref, y1_ref, dinv_ref, w1_ref, b1_ref, y2_ref):
    dinv = dinv_ref[...][:, 0:1]
    agg = (p_ref[0] + p_ref[1] + y1_ref[...]) * dinv
    h = jnp.dot(agg, w1_ref[...], preferred_element_type=jnp.float32) + b1_ref[...]
    y2_ref[...] = jnp.maximum(h, 0.0) * dinv


def _tc_k2(p, y1, dinv16, W1, b1r):
    return pl.pallas_call(
        _tc_k2_body,
        grid=(GRID,),
        in_specs=[
            pl.BlockSpec((NC, BM, D), lambda m: (0, m, 0)),
            pl.BlockSpec((BM, D), lambda m: (m, 0)),
            pl.BlockSpec((BM, 16), lambda m: (m, 0)),
            pl.BlockSpec((D, D), lambda m: (0, 0)),
            pl.BlockSpec((1, D), lambda m: (0, 0)),
        ],
        out_specs=pl.BlockSpec((BM, D), lambda m: (m, 0)),
        out_shape=jax.ShapeDtypeStruct((NPAD, D), jnp.float32),
    )(p, y1, dinv16, W1, b1r)


def _tc_k3_body(q_ref, y2_ref, dinv_ref, w2_ref, w3_ref, b2_ref, b3_ref, z_ref):
    dinv = dinv_ref[...][:, 0:1]
    agg = (q_ref[0] + q_ref[1] + y2_ref[...]) * dinv
    w23 = jnp.dot(w2_ref[...], w3_ref[...], preferred_element_type=jnp.float32)
    zz = jnp.dot(agg, w23, preferred_element_type=jnp.float32)
    crow = jnp.dot(b2_ref[...], w3_ref[...], preferred_element_type=jnp.float32) + b3_ref[...]
    zcol = jax.nn.sigmoid(zz[:, 0:1] + crow[:, 0:1])
    z_ref[...] = jnp.broadcast_to(zcol, (BM, D))


def _tc_k3(q, y2, dinv16, W2, W3p, b2r, b3r):
    return pl.pallas_call(
        _tc_k3_body,
        grid=(GRID,),
        in_specs=[
            pl.BlockSpec((NC, BM, D), lambda m: (0, m, 0)),
            pl.BlockSpec((BM, D), lambda m: (m, 0)),
            pl.BlockSpec((BM, 16), lambda m: (m, 0)),
            pl.BlockSpec((D, 2 * D), lambda m: (0, 0)),
            pl.BlockSpec((2 * D, D), lambda m: (0, 0)),
            pl.BlockSpec((1, 2 * D), lambda m: (0, 0)),
            pl.BlockSpec((1, D), lambda m: (0, 0)),
        ],
        out_specs=pl.BlockSpec((BM, D), lambda m: (m, 0)),
        out_shape=jax.ShapeDtypeStruct((NPAD, D), jnp.float32),
    )(q, y2, dinv16, W2, W3p, b2r, b3r)


def kernel(x, edge_index, join_index, W1, b1, W2, b2, W3, b3):
    src = edge_index[0].astype(jnp.int32)
    dst = edge_index[1].astype(jnp.int32)
    pad = EPAD - E
    src = jnp.concatenate([src, jnp.full((pad,), N, jnp.int32)])
    dst = jnp.concatenate([dst, jnp.full((pad,), N, jnp.int32)])
    src_idx = src.reshape(NW, CH, CHUNK)
    dst_idx = dst.reshape(NW, CH, CHUNK)
    jidx = join_index.astype(jnp.int32).reshape(NW, BJW)

    x_pad = jnp.pad(x, ((0, NPAD - N), (0, 0)))
    zeros16 = jnp.zeros((NPAD, 16), jnp.float32)
    zerosD = jnp.zeros((NPAD, D), jnp.float32)
    ones = jnp.ones((CHUNK, 16), jnp.float32)
    b1r = b1.reshape(1, D)
    b2r = b2.reshape(1, 2 * D)
    W3p = jnp.pad(W3, ((0, 0), (0, D - 1)))
    b3r = jnp.broadcast_to(b3.reshape(1, 1), (1, D)).astype(jnp.float32)

    deg_part = _sc_deg(dst_idx, ones, zeros16)
    y1, dinv16 = _tc_k1(deg_part, x_pad)
    p = _sc_scatter(y1, src_idx, dst_idx, zerosD)
    y2 = _tc_k2(p, y1, dinv16, W1, b1r)
    q = _sc_scatter(y2, src_idx, dst_idx, zerosD)
    z16 = _tc_k3(q, y2, dinv16, W2, W3p, b2r, b3r)
    zj = _sc_join(z16, jidx)
    return zj[:, :1]
